# Initial kernel scaffold; baseline (speedup 1.0000x reference)
import jax
import jax.numpy as jnp
from jax.experimental import pallas as pl


def kernel(x, W):
    raise NotImplementedError("write your pallas kernel here")


# speedup vs baseline: 1.9079x; 1.9079x over previous
import functools
import math

import jax
import jax.numpy as jnp
from jax import lax
from jax.experimental import pallas as pl
from jax.experimental.pallas import tpu as pltpu
from jax.experimental.pallas import tpu_sc as plsc

_N = 4096
_NUM_BUCKETS = 32
_SCALE = 0.125
_NSHIFT = 16
_VLEN = 8192


def _v16_body(w_ref, v_ref):
    s = lax.broadcasted_iota(jnp.int32, (_NSHIFT, _VLEN), 0)
    m = lax.broadcasted_iota(jnp.int32, (_NSHIFT, _VLEN), 1)
    d = jnp.minimum(m + s, 2 * _N - 2)
    rel_pos = d - (_N - 1)
    n = -rel_pos
    ret = (n < 0).astype(jnp.int32) * 16
    n = jnp.abs(n)
    is_small = n < 8
    safe_n = jnp.maximum(n, 1).astype(jnp.float32)
    val_if_large = 8 + (jnp.log(safe_n / 8) / math.log(16.0) * 8).astype(jnp.int32)
    val_if_large = jnp.minimum(val_if_large, 15)
    bucket = ret + jnp.where(is_small, n, val_if_large)
    acc = jnp.full((_NSHIFT, _VLEN), w_ref[0, 0], jnp.float32)
    for b in range(1, _NUM_BUCKETS):
        acc = jnp.where(bucket == b, w_ref[b, 0], acc)
    v_ref[...] = acc * _SCALE


def _compute_v16(W):
    return pl.pallas_call(
        _v16_body,
        in_specs=[pl.BlockSpec(memory_space=pltpu.SMEM)],
        out_specs=pl.BlockSpec(memory_space=pltpu.VMEM),
        out_shape=jax.ShapeDtypeStruct((_NSHIFT, _VLEN), jnp.float32),
    )(W)


_MESH = plsc.VectorSubcoreMesh(core_axis_name="c", subcore_axis_name="s")


@functools.partial(
    pl.kernel,
    mesh=_MESH,
    out_type=jax.ShapeDtypeStruct((_N, _N), jnp.float32),
    scratch_types=[pltpu.VMEM((_VLEN,), jnp.float32)],
    compiler_params=pltpu.CompilerParams(use_tc_tiling_on_sc=False),
)
def _expand(v16_hbm, out_hbm, vrow):
    cid = lax.axis_index("c")
    sid = lax.axis_index("s")
    pltpu.sync_copy(v16_hbm.at[sid], vrow)

    def body(k, carry):
        t = k + 128 * cid
        i = (15 - sid) + 16 * t
        a = 16 * (255 - t)
        pltpu.sync_copy(vrow.at[pl.ds(a, _N)], out_hbm.at[i])
        return carry

    lax.fori_loop(0, 128, body, 0)


def kernel(x, W):
    del x
    return _expand(_compute_v16(W))


# speedup vs baseline: 1.9211x; 1.0069x over previous
import functools

import jax
import jax.numpy as jnp
from jax import lax
from jax.experimental import pallas as pl
from jax.experimental.pallas import tpu as pltpu
from jax.experimental.pallas import tpu_sc as plsc

_N = 4096
_SCALE = 0.125
_VLEN = 8192
_L = 16
_NCHUNK = _VLEN // _L
_LO = 249
_HI = 263


def _bucket_values(d, wtab):
    rel_pos = d - (_N - 1)
    n = -rel_pos
    ret = (n < 0).astype(jnp.int32) * 16
    n = jnp.abs(n)
    is_small = n < 8
    safe_n = jnp.maximum(n, 1)
    e = (lax.bitcast_convert_type(safe_n.astype(jnp.float32), jnp.int32) >> 23) - 127
    val_if_large = 2 * e + 2 + (safe_n * safe_n >= (1 << (2 * e + 1))).astype(jnp.int32)
    val_if_large = jnp.minimum(val_if_large, 15)
    bucket = ret + jnp.where(is_small, n, val_if_large)
    return plsc.load_gather(wtab, [bucket]) * _SCALE


_MESH = plsc.VectorSubcoreMesh(core_axis_name="c", subcore_axis_name="s")


@functools.partial(
    pl.kernel,
    mesh=_MESH,
    out_type=jax.ShapeDtypeStruct((_N, _N), jnp.float32),
    scratch_types=[
        pltpu.VMEM((_VLEN,), jnp.float32),
        pltpu.VMEM((32,), jnp.float32),
    ],
    compiler_params=pltpu.CompilerParams(
        use_tc_tiling_on_sc=False, needs_layout_passes=False
    ),
)
def _sc_bias(w_hbm, out_hbm, vrow, wtab):
    cid = lax.axis_index("c")
    sid = lax.axis_index("s")
    pltpu.sync_copy(w_hbm, wtab)

    c15 = plsc.load_gather(wtab, [jnp.full((_L,), 15, jnp.int32)]) * _SCALE
    c31 = plsc.load_gather(wtab, [jnp.full((_L,), 31, jnp.int32)]) * _SCALE
    lane = lax.iota(jnp.int32, _L)

    def fill_lo(k, carry):
        vrow[pl.ds(_L * k, _L)] = c15
        return carry

    def fill_mid(k, carry):
        d = _L * k + lane + sid
        vrow[pl.ds(_L * k, _L)] = _bucket_values(d, wtab)
        return carry

    def fill_hi(k, carry):
        vrow[pl.ds(_L * k, _L)] = c31
        return carry

    lax.fori_loop(0, _LO, fill_lo, 0)
    lax.fori_loop(_LO, _HI, fill_mid, 0)
    lax.fori_loop(_HI, _NCHUNK, fill_hi, 0)

    def write_row(k, carry):
        t = k + 128 * cid
        i = (15 - sid) + 16 * t
        a = 16 * (255 - t)
        pltpu.sync_copy(vrow.at[pl.ds(a, _N)], out_hbm.at[i])
        return carry

    lax.fori_loop(0, 128, write_row, 0)


def kernel(x, W):
    del x
    return _sc_bias(W.reshape(32))


# speedup vs baseline: 3.6852x; 1.9182x over previous
import functools

import jax
import jax.numpy as jnp
from jax import lax
from jax.experimental import pallas as pl
from jax.experimental.pallas import tpu as pltpu
from jax.experimental.pallas import tpu_sc as plsc

_N = 4096
_SCALE = 0.125
_L = 16
_NROWBLK = 64
_NCHUNK = _NROWBLK * 128 // _L
_LO = 241
_HI = 271


def _bucket_values(d, wtab):
    rel_pos = d - (_N - 1)
    n = -rel_pos
    ret = (n < 0).astype(jnp.int32) * 16
    n = jnp.abs(n)
    is_small = n < 8
    safe_n = jnp.maximum(n, 1)
    e = (lax.bitcast_convert_type(safe_n.astype(jnp.float32), jnp.int32) >> 23) - 127
    val_if_large = 2 * e + 2 + (safe_n * safe_n >= (1 << (2 * e + 1))).astype(jnp.int32)
    val_if_large = jnp.minimum(val_if_large, 15)
    bucket = ret + jnp.where(is_small, n, val_if_large)
    return plsc.load_gather(wtab, [bucket]) * _SCALE


_MESH = plsc.VectorSubcoreMesh(core_axis_name="c", subcore_axis_name="s")


@functools.partial(
    pl.kernel,
    mesh=_MESH,
    out_type=jax.ShapeDtypeStruct((_N // 8, _N // 128, 8, 128), jnp.float32),
    scratch_types=[
        pltpu.VMEM((8, _NROWBLK, 128), jnp.float32),
        pltpu.VMEM((32,), jnp.float32),
    ],
    compiler_params=pltpu.CompilerParams(
        use_tc_tiling_on_sc=False, needs_layout_passes=False
    ),
)
def _sc_bias(w_hbm, out_hbm, vshift, wtab):
    cid = lax.axis_index("c")
    sid = lax.axis_index("s")
    pltpu.sync_copy(w_hbm, wtab)

    c15 = plsc.load_gather(wtab, [jnp.full((_L,), 15, jnp.int32)]) * _SCALE
    c31 = plsc.load_gather(wtab, [jnp.full((_L,), 31, jnp.int32)]) * _SCALE
    lane = lax.iota(jnp.int32, _L)

    def fill(u, k, val):
        vshift[u, k // 8, pl.ds(_L * (k % 8), _L)] = val

    def fill_u(u, carry):
        def fill_lo(k, c):
            fill(u, k, c15)
            return c

        def fill_mid(k, c):
            d = _L * k + lane + sid + 16 * u
            fill(u, k, _bucket_values(d, wtab))
            return c

        def fill_hi(k, c):
            fill(u, k, c31)
            return c

        lax.fori_loop(0, _LO, fill_lo, 0)
        lax.fori_loop(_LO, _HI, fill_mid, 0)
        lax.fori_loop(_HI, _NCHUNK, fill_hi, 0)
        return carry

    lax.fori_loop(0, 8, fill_u, 0)

    def write_rows(u, carry):
        sprime = sid + 16 * u

        def write_row(k, c):
            m = k + 16 * cid
            i = (_N - 1) - sprime - 128 * m
            pltpu.sync_copy(
                vshift.at[u, pl.ds(m, 32), :],
                out_hbm.at[i // 8, :, i % 8],
            )
            return c

        lax.fori_loop(0, 16, write_row, 0)
        return carry

    lax.fori_loop(0, 8, write_rows, 0)


def kernel(x, W):
    del x
    o4 = _sc_bias(W.reshape(32))
    return o4.transpose(0, 2, 1, 3).reshape(_N, _N)


# speedup vs baseline: 5.3459x; 1.4506x over previous
import functools

import jax
import jax.numpy as jnp
from jax import lax
from jax.experimental import pallas as pl
from jax.experimental.pallas import tpu as pltpu
from jax.experimental.pallas import tpu_sc as plsc

_N = 4096
_SCALE = 0.125
_L = 16
_NROWBLK = 64
_NCHUNK = _NROWBLK * 128 // _L
_LO = 241
_HI = 271


def _bucket_values(d, wtab):
    rel_pos = d - (_N - 1)
    n = -rel_pos
    ret = (n < 0).astype(jnp.int32) * 16
    n = jnp.abs(n)
    is_small = n < 8
    safe_n = jnp.maximum(n, 1)
    e = (lax.bitcast_convert_type(safe_n.astype(jnp.float32), jnp.int32) >> 23) - 127
    val_if_large = 2 * e + 2 + (safe_n * safe_n >= (1 << (2 * e + 1))).astype(jnp.int32)
    val_if_large = jnp.minimum(val_if_large, 15)
    bucket = ret + jnp.where(is_small, n, val_if_large)
    return plsc.load_gather(wtab, [bucket]) * _SCALE


_MESH = plsc.VectorSubcoreMesh(core_axis_name="c", subcore_axis_name="s")


@functools.partial(
    pl.kernel,
    mesh=_MESH,
    out_type=jax.ShapeDtypeStruct((_N // 8, _N // 128, 8, 128), jnp.float32),
    scratch_types=[
        pltpu.VMEM((8, _NROWBLK, 128), jnp.float32),
        pltpu.VMEM((32,), jnp.float32),
        pltpu.SemaphoreType.DMA,
    ],
    compiler_params=pltpu.CompilerParams(
        use_tc_tiling_on_sc=False, needs_layout_passes=False
    ),
)
def _sc_bias(w_hbm, out_hbm, vshift, wtab, sem):
    cid = lax.axis_index("c")
    sid = lax.axis_index("s")
    pltpu.sync_copy(w_hbm, wtab)

    c15 = plsc.load_gather(wtab, [jnp.full((_L,), 15, jnp.int32)]) * _SCALE
    c31 = plsc.load_gather(wtab, [jnp.full((_L,), 31, jnp.int32)]) * _SCALE
    lane = lax.iota(jnp.int32, _L)

    def fill(u, k, val):
        vshift[u, k // 8, pl.ds(_L * (k % 8), _L)] = val

    def row_copy(u, k):
        m = k + 16 * cid
        i = (_N - 1) - (sid + 16 * u) - 128 * m
        return pltpu.make_async_copy(
            vshift.at[u, pl.ds(m, 32), :],
            out_hbm.at[i // 8, :, i % 8],
            sem,
        )

    def fill_u(u, carry):
        def fill_lo(k, c):
            fill(u, k, c15)
            return c

        def fill_mid(k, c):
            d = _L * k + lane + sid + 16 * u
            fill(u, k, _bucket_values(d, wtab))
            return c

        def fill_hi(k, c):
            fill(u, k, c31)
            return c

        lax.fori_loop(0, _LO, fill_lo, 0)
        lax.fori_loop(_LO, _HI, fill_mid, 0)
        lax.fori_loop(_HI, _NCHUNK, fill_hi, 0)

        def fire_row(k, c):
            row_copy(u, k).start()
            return c

        lax.fori_loop(0, 16, fire_row, 0)
        return carry

    lax.fori_loop(0, 8, fill_u, 0)

    def drain_u(u, carry):
        def drain_row(k, c):
            row_copy(u, k).wait()
            return c

        lax.fori_loop(0, 16, drain_row, 0)
        return carry

    lax.fori_loop(0, 8, drain_u, 0)


def kernel(x, W):
    del x
    o4 = _sc_bias(W.reshape(32))
    return o4.transpose(0, 2, 1, 3).reshape(_N, _N)
